# baseline (device time: 390587 ns/iter reference)
import jax
import jax.numpy as jnp
from jax import lax
from jax.experimental import pallas as pl
from jax.experimental.pallas import tpu as pltpu

M = 4096
D = 4096
F_SHARD = 8192
M_LOCAL = M // 2

BN = 2048
BK = 512
N_J = D // BN
N_K = F_SHARD // BK

S = 16
ROWS = M_LOCAL // S


def _mm_body(y_ref, dy_ref, w_ref, out_ref, acc_ref, bcvt, osems):
    k = pl.program_id(0)
    j = pl.program_id(1)

    @pl.when(k == 0)
    def _():
        acc_ref[j] = jnp.zeros_like(acc_ref[j])

    a = dy_ref[...].astype(jnp.bfloat16)
    b = w_ref[...].astype(jnp.bfloat16)
    acc_ref[j] += lax.dot_general(
        a, b, (((1,), (1,)), ((), ())), preferred_element_type=jnp.float32
    )

    @pl.when(k == N_K - 1)
    def _():
        @pl.when(j == 1)
        def _():
            pltpu.make_async_copy(
                bcvt.at[0], out_ref.at[:, pl.ds(0, BN)], osems.at[0]
            ).wait()
        bcvt[0] = acc_ref[j].astype(jnp.bfloat16)
        cp = pltpu.make_async_copy(
            bcvt.at[0], out_ref.at[:, pl.ds(j * BN, BN)], osems.at[0]
        )
        cp.start()

        @pl.when(j == N_J - 1)
        def _():
            cp.wait()


def _partial_matmul(dy, w, my_y):
    grid_spec = pltpu.PrefetchScalarGridSpec(
        num_scalar_prefetch=1,
        grid=(N_K, N_J),
        in_specs=[
            pl.BlockSpec((M_LOCAL, BK), lambda k, j, y: (y[0], k)),
            pl.BlockSpec((BN, BK), lambda k, j, y: (j, k)),
        ],
        out_specs=pl.BlockSpec(memory_space=pl.ANY),
        scratch_shapes=[
            pltpu.VMEM((N_J, M_LOCAL, BN), jnp.float32),
            pltpu.VMEM((1, M_LOCAL, BN), jnp.bfloat16),
            pltpu.SemaphoreType.DMA((1,)),
        ],
    )
    return pl.pallas_call(
        _mm_body,
        grid_spec=grid_spec,
        out_shape=jax.ShapeDtypeStruct((M_LOCAL, D), jnp.bfloat16),
        compiler_params=pltpu.CompilerParams(
            vmem_limit_bytes=62 * 1024 * 1024,
        ),
    )(my_y, dy, w)


def _comm_body(partial_ref, out_ref, pbuf, xrecv, yrecv,
               x_send_sems, x_recv_sems, y_send_sems, y_recv_sems,
               in_sem, out_own_sems, out_far_sems):
    my_x = lax.axis_index("x")
    my_y = lax.axis_index("y")

    cp_in = pltpu.make_async_copy(partial_ref, pbuf, in_sem)
    cp_in.start()

    barrier_sem = pltpu.get_barrier_semaphore()
    pl.semaphore_signal(barrier_sem, inc=1, device_id=(1 - my_x, my_y),
                        device_id_type=pl.DeviceIdType.MESH)
    pl.semaphore_signal(barrier_sem, inc=1, device_id=(my_x, 1 - my_y),
                        device_id_type=pl.DeviceIdType.MESH)
    pl.semaphore_wait(barrier_sem, 2)
    cp_in.wait()

    x_rdmas = []
    for s in range(S):
        sl = pl.ds(s * ROWS, ROWS)
        r = pltpu.make_async_remote_copy(
            src_ref=pbuf.at[sl],
            dst_ref=xrecv.at[sl],
            send_sem=x_send_sems.at[s],
            recv_sem=x_recv_sems.at[s],
            device_id=(1 - my_x, my_y),
            device_id_type=pl.DeviceIdType.MESH,
        )
        r.start()
        x_rdmas.append(r)

    y_rdmas = []
    out_copies = []
    for s in range(S):
        sl = pl.ds(s * ROWS, ROWS)
        x_rdmas[s].wait()
        pbuf[sl, :] = (pbuf[sl, :].astype(jnp.float32)
                       + xrecv[sl, :].astype(jnp.float32)).astype(jnp.bfloat16)
        r = pltpu.make_async_remote_copy(
            src_ref=pbuf.at[sl],
            dst_ref=yrecv.at[sl],
            send_sem=y_send_sems.at[s],
            recv_sem=y_recv_sems.at[s],
            device_id=(my_x, 1 - my_y),
            device_id_type=pl.DeviceIdType.MESH,
        )
        r.start()
        y_rdmas.append(r)
        cp = pltpu.make_async_copy(
            pbuf.at[sl],
            out_ref.at[pl.ds(my_y * M_LOCAL + s * ROWS, ROWS), :],
            out_own_sems.at[s],
        )
        cp.start()
        out_copies.append(cp)

    for s in range(S):
        sl = pl.ds(s * ROWS, ROWS)
        y_rdmas[s].wait()
        cp = pltpu.make_async_copy(
            yrecv.at[sl],
            out_ref.at[pl.ds((1 - my_y) * M_LOCAL + s * ROWS, ROWS), :],
            out_far_sems.at[s],
        )
        cp.start()
        out_copies.append(cp)

    for cp in out_copies:
        cp.wait()


def _allreduce_gather(partial):
    return pl.pallas_call(
        _comm_body,
        out_shape=jax.ShapeDtypeStruct((M, D), jnp.bfloat16),
        in_specs=[pl.BlockSpec(memory_space=pl.ANY)],
        out_specs=pl.BlockSpec(memory_space=pl.ANY),
        scratch_shapes=[
            pltpu.VMEM((M_LOCAL, D), jnp.bfloat16),
            pltpu.VMEM((M_LOCAL, D), jnp.bfloat16),
            pltpu.VMEM((M_LOCAL, D), jnp.bfloat16),
            pltpu.SemaphoreType.DMA((S,)),
            pltpu.SemaphoreType.DMA((S,)),
            pltpu.SemaphoreType.DMA((S,)),
            pltpu.SemaphoreType.DMA((S,)),
            pltpu.SemaphoreType.DMA,
            pltpu.SemaphoreType.DMA((S,)),
            pltpu.SemaphoreType.DMA((S,)),
        ],
        compiler_params=pltpu.CompilerParams(
            collective_id=0,
            vmem_limit_bytes=62 * 1024 * 1024,
        ),
    )(partial)


def kernel(dy, W):
    import os
    part = os.environ.get("KERNEL_PART", "")
    my_y = lax.axis_index("y")
    y_scalar = jnp.reshape(my_y, (1,)).astype(jnp.int32)
    if part == "mm":
        partial = _partial_matmul(dy, W, y_scalar)
        return jnp.concatenate([partial, partial], 0).astype(jnp.float32)
    if part == "comm":
        partial = dy[:M_LOCAL, :D].astype(jnp.bfloat16)
        return _allreduce_gather(partial)
    partial = _partial_matmul(dy, W, y_scalar)
    return _allreduce_gather(partial)


# device time: 305571 ns/iter; 1.2782x vs baseline; 1.2782x over previous
import jax
import jax.numpy as jnp
from jax import lax
from jax.experimental import pallas as pl
from jax.experimental.pallas import tpu as pltpu

M = 4096
D = 4096
F_SHARD = 8192
M_LOCAL = M // 2

BN = 2048
BK = 1024
N_J = D // BN
N_K = F_SHARD // BK

S = 16
ROWS = M_LOCAL // S


def _mm_body(y_ref, dy_ref, w_ref, out_ref, acc_ref, bcvt, osems):
    j = pl.program_id(0)
    k = pl.program_id(1)

    @pl.when(k == 0)
    def _():
        acc_ref[...] = jnp.zeros_like(acc_ref)

    a = dy_ref[...].astype(jnp.bfloat16)
    b = w_ref[...].astype(jnp.bfloat16)
    acc_ref[...] += lax.dot_general(
        a, b, (((1,), (1,)), ((), ())), preferred_element_type=jnp.float32
    )

    @pl.when(k == N_K - 1)
    def _():
        @pl.when(j == 1)
        def _():
            pltpu.make_async_copy(
                bcvt.at[0], out_ref.at[:, pl.ds(0, BN)], osems.at[0]
            ).wait()
        bcvt[0] = acc_ref[...].astype(jnp.bfloat16)
        cp = pltpu.make_async_copy(
            bcvt.at[0], out_ref.at[:, pl.ds(j * BN, BN)], osems.at[0]
        )
        cp.start()

        @pl.when(j == N_J - 1)
        def _():
            cp.wait()


def _partial_matmul(dy, w, my_y):
    grid_spec = pltpu.PrefetchScalarGridSpec(
        num_scalar_prefetch=1,
        grid=(N_J, N_K),
        in_specs=[
            pl.BlockSpec((M_LOCAL, BK), lambda j, k, y: (y[0], k)),
            pl.BlockSpec((BN, BK), lambda j, k, y: (j, k)),
        ],
        out_specs=pl.BlockSpec(memory_space=pl.ANY),
        scratch_shapes=[
            pltpu.VMEM((M_LOCAL, BN), jnp.float32),
            pltpu.VMEM((1, M_LOCAL, BN), jnp.bfloat16),
            pltpu.SemaphoreType.DMA((1,)),
        ],
    )
    return pl.pallas_call(
        _mm_body,
        grid_spec=grid_spec,
        out_shape=jax.ShapeDtypeStruct((M_LOCAL, D), jnp.bfloat16),
        compiler_params=pltpu.CompilerParams(
            vmem_limit_bytes=67_000_000,
        ),
    )(my_y, dy, w)


def _comm_body(partial_ref, out_ref, pbuf, xrecv, yrecv,
               x_send_sems, x_recv_sems, y_send_sems, y_recv_sems,
               in_sem, out_own_sems, out_far_sems):
    my_x = lax.axis_index("x")
    my_y = lax.axis_index("y")

    cp_in = pltpu.make_async_copy(partial_ref, pbuf, in_sem)
    cp_in.start()

    barrier_sem = pltpu.get_barrier_semaphore()
    pl.semaphore_signal(barrier_sem, inc=1, device_id=(1 - my_x, my_y),
                        device_id_type=pl.DeviceIdType.MESH)
    pl.semaphore_signal(barrier_sem, inc=1, device_id=(my_x, 1 - my_y),
                        device_id_type=pl.DeviceIdType.MESH)
    pl.semaphore_wait(barrier_sem, 2)
    cp_in.wait()

    x_rdmas = []
    for s in range(S):
        sl = pl.ds(s * ROWS, ROWS)
        r = pltpu.make_async_remote_copy(
            src_ref=pbuf.at[sl],
            dst_ref=xrecv.at[sl],
            send_sem=x_send_sems.at[s],
            recv_sem=x_recv_sems.at[s],
            device_id=(1 - my_x, my_y),
            device_id_type=pl.DeviceIdType.MESH,
        )
        r.start()
        x_rdmas.append(r)

    y_rdmas = []
    out_copies = []
    for s in range(S):
        sl = pl.ds(s * ROWS, ROWS)
        x_rdmas[s].wait()
        pbuf[sl, :] = (pbuf[sl, :].astype(jnp.float32)
                       + xrecv[sl, :].astype(jnp.float32)).astype(jnp.bfloat16)
        r = pltpu.make_async_remote_copy(
            src_ref=pbuf.at[sl],
            dst_ref=yrecv.at[sl],
            send_sem=y_send_sems.at[s],
            recv_sem=y_recv_sems.at[s],
            device_id=(my_x, 1 - my_y),
            device_id_type=pl.DeviceIdType.MESH,
        )
        r.start()
        y_rdmas.append(r)
        cp = pltpu.make_async_copy(
            pbuf.at[sl],
            out_ref.at[pl.ds(my_y * M_LOCAL + s * ROWS, ROWS), :],
            out_own_sems.at[s],
        )
        cp.start()
        out_copies.append(cp)

    for s in range(S):
        sl = pl.ds(s * ROWS, ROWS)
        y_rdmas[s].wait()
        cp = pltpu.make_async_copy(
            yrecv.at[sl],
            out_ref.at[pl.ds((1 - my_y) * M_LOCAL + s * ROWS, ROWS), :],
            out_far_sems.at[s],
        )
        cp.start()
        out_copies.append(cp)

    for cp in out_copies:
        cp.wait()


def _allreduce_gather(partial):
    return pl.pallas_call(
        _comm_body,
        out_shape=jax.ShapeDtypeStruct((M, D), jnp.bfloat16),
        in_specs=[pl.BlockSpec(memory_space=pl.ANY)],
        out_specs=pl.BlockSpec(memory_space=pl.ANY),
        scratch_shapes=[
            pltpu.VMEM((M_LOCAL, D), jnp.bfloat16),
            pltpu.VMEM((M_LOCAL, D), jnp.bfloat16),
            pltpu.VMEM((M_LOCAL, D), jnp.bfloat16),
            pltpu.SemaphoreType.DMA((S,)),
            pltpu.SemaphoreType.DMA((S,)),
            pltpu.SemaphoreType.DMA((S,)),
            pltpu.SemaphoreType.DMA((S,)),
            pltpu.SemaphoreType.DMA,
            pltpu.SemaphoreType.DMA((S,)),
            pltpu.SemaphoreType.DMA((S,)),
        ],
        compiler_params=pltpu.CompilerParams(
            collective_id=0,
            vmem_limit_bytes=62 * 1024 * 1024,
        ),
    )(partial)


FB = 1024
FN_J = D // FB
FBK = 512
FN_K = F_SHARD // FBK


def _fused_body(y_ref, dy_ref, w_ref, out_ref, acc, pbuf, xrecv,
                x_send, x_recv, y_send, y_recv, own_sems):
    j = pl.program_id(0)
    k = pl.program_id(1)
    my_x = lax.axis_index("x")
    my_y = lax.axis_index("y")

    @pl.when((j == 0) & (k == 0))
    def _():
        barrier_sem = pltpu.get_barrier_semaphore()
        pl.semaphore_signal(barrier_sem, inc=1, device_id=(1 - my_x, my_y),
                            device_id_type=pl.DeviceIdType.MESH)
        pl.semaphore_signal(barrier_sem, inc=1, device_id=(my_x, 1 - my_y),
                            device_id_type=pl.DeviceIdType.MESH)
        pl.semaphore_wait(barrier_sem, 2)

    @pl.when(k == 0)
    def _():
        acc[...] = jnp.zeros_like(acc)

    a = dy_ref[...].astype(jnp.bfloat16)
    b = w_ref[...].astype(jnp.bfloat16)
    acc[...] += lax.dot_general(
        a, b, (((1,), (1,)), ((), ())), preferred_element_type=jnp.float32
    )

    def csl(p):
        return (slice(None), pl.ds(p * FB, FB))

    def x_desc(p):
        return pltpu.make_async_remote_copy(
            src_ref=pbuf.at[csl(p)],
            dst_ref=xrecv.at[csl(p)],
            send_sem=x_send.at[p],
            recv_sem=x_recv.at[p],
            device_id=(1 - my_x, my_y),
            device_id_type=pl.DeviceIdType.MESH,
        )

    def y_desc(p):
        return pltpu.make_async_remote_copy(
            src_ref=pbuf.at[csl(p)],
            dst_ref=out_ref.at[pl.ds(my_y * M_LOCAL, M_LOCAL),
                               pl.ds(p * FB, FB)],
            send_sem=y_send.at[p],
            recv_sem=y_recv.at[p],
            device_id=(my_x, 1 - my_y),
            device_id_type=pl.DeviceIdType.MESH,
        )

    def own_desc(p):
        return pltpu.make_async_copy(
            pbuf.at[csl(p)],
            out_ref.at[pl.ds(my_y * M_LOCAL, M_LOCAL), pl.ds(p * FB, FB)],
            own_sems.at[p],
        )

    def process(p):
        x_desc(p).wait()
        pbuf[csl(p)] = (pbuf[csl(p)].astype(jnp.float32)
                        + xrecv[csl(p)].astype(jnp.float32)
                        ).astype(jnp.bfloat16)
        y_desc(p).start()
        own_desc(p).start()

    for jj in range(FN_J):
        @pl.when((j == jj) & (k == FN_K - 1))
        def _(jj=jj):
            pbuf[csl(jj)] = acc[...].astype(jnp.bfloat16)
            x_desc(jj).start()
            if jj >= 1:
                process(jj - 1)
            if jj == FN_J - 1:
                process(jj)
                for p in range(FN_J):
                    y_desc(p).wait()
                    own_desc(p).wait()


def _fused(dy, w, my_y):
    grid_spec = pltpu.PrefetchScalarGridSpec(
        num_scalar_prefetch=1,
        grid=(FN_J, FN_K),
        in_specs=[
            pl.BlockSpec((M_LOCAL, FBK), lambda j, k, y: (y[0], k)),
            pl.BlockSpec((FB, FBK), lambda j, k, y: (j, k)),
        ],
        out_specs=pl.BlockSpec(memory_space=pl.ANY),
        scratch_shapes=[
            pltpu.VMEM((M_LOCAL, FB), jnp.float32),
            pltpu.VMEM((M_LOCAL, D), jnp.bfloat16),
            pltpu.VMEM((M_LOCAL, D), jnp.bfloat16),
            pltpu.SemaphoreType.DMA((FN_J,)),
            pltpu.SemaphoreType.DMA((FN_J,)),
            pltpu.SemaphoreType.DMA((FN_J,)),
            pltpu.SemaphoreType.DMA((FN_J,)),
            pltpu.SemaphoreType.DMA((FN_J,)),
        ],
    )
    return pl.pallas_call(
        _fused_body,
        grid_spec=grid_spec,
        out_shape=jax.ShapeDtypeStruct((M, D), jnp.bfloat16),
        compiler_params=pltpu.CompilerParams(
            collective_id=0,
            vmem_limit_bytes=67_000_000,
        ),
    )(my_y, dy, w)


def kernel(dy, W):
    import os
    part = os.environ.get("KERNEL_PART", "")
    my_y = lax.axis_index("y")
    y_scalar = jnp.reshape(my_y, (1,)).astype(jnp.int32)
    if part == "mm":
        partial = _partial_matmul(dy, W, y_scalar)
        return jnp.concatenate([partial, partial], 0).astype(jnp.float32)
    if part == "comm":
        partial = dy[:M_LOCAL, :D].astype(jnp.bfloat16)
        return _allreduce_gather(partial)
    if part == "twostage":
        partial = _partial_matmul(dy, W, y_scalar)
        return _allreduce_gather(partial)
    return _fused(dy, W, y_scalar)


# device time: 305424 ns/iter; 1.2788x vs baseline; 1.0005x over previous
import jax
import jax.numpy as jnp
from jax import lax
from jax.experimental import pallas as pl
from jax.experimental.pallas import tpu as pltpu

M = 4096
D = 4096
F_SHARD = 8192
M_LOCAL = M // 2

BN = 2048
BK = 1024
N_J = D // BN
N_K = F_SHARD // BK

S = 16
ROWS = M_LOCAL // S


def _mm_body(y_ref, dy_ref, w_ref, out_ref, acc_ref, bcvt, osems):
    j = pl.program_id(0)
    k = pl.program_id(1)

    @pl.when(k == 0)
    def _():
        acc_ref[...] = jnp.zeros_like(acc_ref)

    a = dy_ref[...].astype(jnp.bfloat16)
    b = w_ref[...].astype(jnp.bfloat16)
    acc_ref[...] += lax.dot_general(
        a, b, (((1,), (1,)), ((), ())), preferred_element_type=jnp.float32
    )

    @pl.when(k == N_K - 1)
    def _():
        @pl.when(j == 1)
        def _():
            pltpu.make_async_copy(
                bcvt.at[0], out_ref.at[:, pl.ds(0, BN)], osems.at[0]
            ).wait()
        bcvt[0] = acc_ref[...].astype(jnp.bfloat16)
        cp = pltpu.make_async_copy(
            bcvt.at[0], out_ref.at[:, pl.ds(j * BN, BN)], osems.at[0]
        )
        cp.start()

        @pl.when(j == N_J - 1)
        def _():
            cp.wait()


def _partial_matmul(dy, w, my_y):
    grid_spec = pltpu.PrefetchScalarGridSpec(
        num_scalar_prefetch=1,
        grid=(N_J, N_K),
        in_specs=[
            pl.BlockSpec((M_LOCAL, BK), lambda j, k, y: (y[0], k)),
            pl.BlockSpec((BN, BK), lambda j, k, y: (j, k)),
        ],
        out_specs=pl.BlockSpec(memory_space=pl.ANY),
        scratch_shapes=[
            pltpu.VMEM((M_LOCAL, BN), jnp.float32),
            pltpu.VMEM((1, M_LOCAL, BN), jnp.bfloat16),
            pltpu.SemaphoreType.DMA((1,)),
        ],
    )
    return pl.pallas_call(
        _mm_body,
        grid_spec=grid_spec,
        out_shape=jax.ShapeDtypeStruct((M_LOCAL, D), jnp.bfloat16),
        compiler_params=pltpu.CompilerParams(
            vmem_limit_bytes=67_000_000,
        ),
    )(my_y, dy, w)


def _comm_body(partial_ref, out_ref, pbuf, xrecv, yrecv,
               x_send_sems, x_recv_sems, y_send_sems, y_recv_sems,
               in_sem, out_own_sems, out_far_sems):
    my_x = lax.axis_index("x")
    my_y = lax.axis_index("y")

    cp_in = pltpu.make_async_copy(partial_ref, pbuf, in_sem)
    cp_in.start()

    barrier_sem = pltpu.get_barrier_semaphore()
    pl.semaphore_signal(barrier_sem, inc=1, device_id=(1 - my_x, my_y),
                        device_id_type=pl.DeviceIdType.MESH)
    pl.semaphore_signal(barrier_sem, inc=1, device_id=(my_x, 1 - my_y),
                        device_id_type=pl.DeviceIdType.MESH)
    pl.semaphore_wait(barrier_sem, 2)
    cp_in.wait()

    x_rdmas = []
    for s in range(S):
        sl = pl.ds(s * ROWS, ROWS)
        r = pltpu.make_async_remote_copy(
            src_ref=pbuf.at[sl],
            dst_ref=xrecv.at[sl],
            send_sem=x_send_sems.at[s],
            recv_sem=x_recv_sems.at[s],
            device_id=(1 - my_x, my_y),
            device_id_type=pl.DeviceIdType.MESH,
        )
        r.start()
        x_rdmas.append(r)

    y_rdmas = []
    out_copies = []
    for s in range(S):
        sl = pl.ds(s * ROWS, ROWS)
        x_rdmas[s].wait()
        pbuf[sl, :] = (pbuf[sl, :].astype(jnp.float32)
                       + xrecv[sl, :].astype(jnp.float32)).astype(jnp.bfloat16)
        r = pltpu.make_async_remote_copy(
            src_ref=pbuf.at[sl],
            dst_ref=yrecv.at[sl],
            send_sem=y_send_sems.at[s],
            recv_sem=y_recv_sems.at[s],
            device_id=(my_x, 1 - my_y),
            device_id_type=pl.DeviceIdType.MESH,
        )
        r.start()
        y_rdmas.append(r)
        cp = pltpu.make_async_copy(
            pbuf.at[sl],
            out_ref.at[pl.ds(my_y * M_LOCAL + s * ROWS, ROWS), :],
            out_own_sems.at[s],
        )
        cp.start()
        out_copies.append(cp)

    for s in range(S):
        sl = pl.ds(s * ROWS, ROWS)
        y_rdmas[s].wait()
        cp = pltpu.make_async_copy(
            yrecv.at[sl],
            out_ref.at[pl.ds((1 - my_y) * M_LOCAL + s * ROWS, ROWS), :],
            out_far_sems.at[s],
        )
        cp.start()
        out_copies.append(cp)

    for cp in out_copies:
        cp.wait()


def _allreduce_gather(partial):
    return pl.pallas_call(
        _comm_body,
        out_shape=jax.ShapeDtypeStruct((M, D), jnp.bfloat16),
        in_specs=[pl.BlockSpec(memory_space=pl.ANY)],
        out_specs=pl.BlockSpec(memory_space=pl.ANY),
        scratch_shapes=[
            pltpu.VMEM((M_LOCAL, D), jnp.bfloat16),
            pltpu.VMEM((M_LOCAL, D), jnp.bfloat16),
            pltpu.VMEM((M_LOCAL, D), jnp.bfloat16),
            pltpu.SemaphoreType.DMA((S,)),
            pltpu.SemaphoreType.DMA((S,)),
            pltpu.SemaphoreType.DMA((S,)),
            pltpu.SemaphoreType.DMA((S,)),
            pltpu.SemaphoreType.DMA,
            pltpu.SemaphoreType.DMA((S,)),
            pltpu.SemaphoreType.DMA((S,)),
        ],
        compiler_params=pltpu.CompilerParams(
            collective_id=0,
            vmem_limit_bytes=62 * 1024 * 1024,
        ),
    )(partial)


FB = 1024
FN_J = D // FB
FBK = 512
FN_K = F_SHARD // FBK


def _fused_body(y_ref, dy_ref, w_ref, out_ref, acc, pbuf, xrecv,
                x_send, x_recv, y_send, y_recv, own_sems):
    j = pl.program_id(0)
    k = pl.program_id(1)
    my_x = lax.axis_index("x")
    my_y = lax.axis_index("y")

    @pl.when((j == 0) & (k == 0))
    def _():
        barrier_sem = pltpu.get_barrier_semaphore()
        pl.semaphore_signal(barrier_sem, inc=1, device_id=(1 - my_x, my_y),
                            device_id_type=pl.DeviceIdType.MESH)
        pl.semaphore_signal(barrier_sem, inc=1, device_id=(my_x, 1 - my_y),
                            device_id_type=pl.DeviceIdType.MESH)
        pl.semaphore_wait(barrier_sem, 2)

    @pl.when(k == 0)
    def _():
        acc[...] = jnp.zeros_like(acc)

    a = dy_ref[...].astype(jnp.bfloat16)
    b = w_ref[...].astype(jnp.bfloat16)
    acc[...] += lax.dot_general(
        a, b, (((1,), (1,)), ((), ())), preferred_element_type=jnp.float32
    )

    def x_desc(p):
        return pltpu.make_async_remote_copy(
            src_ref=pbuf.at[p],
            dst_ref=xrecv.at[p],
            send_sem=x_send.at[p],
            recv_sem=x_recv.at[p],
            device_id=(1 - my_x, my_y),
            device_id_type=pl.DeviceIdType.MESH,
        )

    def y_desc(p):
        return pltpu.make_async_remote_copy(
            src_ref=pbuf.at[p],
            dst_ref=out_ref.at[pl.ds(my_y * M_LOCAL, M_LOCAL),
                               pl.ds(p * FB, FB)],
            send_sem=y_send.at[p],
            recv_sem=y_recv.at[p],
            device_id=(my_x, 1 - my_y),
            device_id_type=pl.DeviceIdType.MESH,
        )

    def own_desc(p):
        return pltpu.make_async_copy(
            pbuf.at[p],
            out_ref.at[pl.ds(my_y * M_LOCAL, M_LOCAL), pl.ds(p * FB, FB)],
            own_sems.at[p],
        )

    def process(p):
        x_desc(p).wait()
        pbuf[p] = (pbuf[p].astype(jnp.float32)
                   + xrecv[p].astype(jnp.float32)).astype(jnp.bfloat16)
        y_desc(p).start()
        own_desc(p).start()

    for jj in range(FN_J):
        @pl.when((j == jj) & (k == FN_K - 1))
        def _(jj=jj):
            pbuf[jj] = acc[...].astype(jnp.bfloat16)
            x_desc(jj).start()
            if jj >= 1:
                process(jj - 1)
            if jj == FN_J - 1:
                process(jj)
                for p in range(FN_J):
                    y_desc(p).wait()
                    own_desc(p).wait()


def _fused(dy, w, my_y):
    grid_spec = pltpu.PrefetchScalarGridSpec(
        num_scalar_prefetch=1,
        grid=(FN_J, FN_K),
        in_specs=[
            pl.BlockSpec((M_LOCAL, FBK), lambda j, k, y: (y[0], k)),
            pl.BlockSpec((FB, FBK), lambda j, k, y: (j, k)),
        ],
        out_specs=pl.BlockSpec(memory_space=pl.ANY),
        scratch_shapes=[
            pltpu.VMEM((M_LOCAL, FB), jnp.float32),
            pltpu.VMEM((FN_J, M_LOCAL, FB), jnp.bfloat16),
            pltpu.VMEM((FN_J, M_LOCAL, FB), jnp.bfloat16),
            pltpu.SemaphoreType.DMA((FN_J,)),
            pltpu.SemaphoreType.DMA((FN_J,)),
            pltpu.SemaphoreType.DMA((FN_J,)),
            pltpu.SemaphoreType.DMA((FN_J,)),
            pltpu.SemaphoreType.DMA((FN_J,)),
        ],
    )
    return pl.pallas_call(
        _fused_body,
        grid_spec=grid_spec,
        out_shape=jax.ShapeDtypeStruct((M, D), jnp.bfloat16),
        compiler_params=pltpu.CompilerParams(
            collective_id=0,
            vmem_limit_bytes=67_000_000,
        ),
    )(my_y, dy, w)


def kernel(dy, W):
    import os
    part = os.environ.get("KERNEL_PART", "")
    my_y = lax.axis_index("y")
    y_scalar = jnp.reshape(my_y, (1,)).astype(jnp.int32)
    if part == "mm":
        partial = _partial_matmul(dy, W, y_scalar)
        return jnp.concatenate([partial, partial], 0).astype(jnp.float32)
    if part == "comm":
        partial = dy[:M_LOCAL, :D].astype(jnp.bfloat16)
        return _allreduce_gather(partial)
    if part == "twostage":
        partial = _partial_matmul(dy, W, y_scalar)
        return _allreduce_gather(partial)
    return _fused(dy, W, y_scalar)
